# baseline (speedup 1.0000x reference)
import jax
import jax.numpy as jnp
from jax.experimental import pallas as pl

KNN = 16
NPTS = 4096
ROWS = 512
DIM = 16


def _knn_kernel(xr_ref, xc_ref, dst_ref, src_ref):
    s = pl.program_id(0)
    r = pl.program_id(1)
    xr = xr_ref[0]
    xc = xc_ref[0]

    x2r = jnp.sum(xr * xr, axis=1, keepdims=True)
    sq_c = xc * xc
    ones = jnp.ones((8, DIM), jnp.float32)
    x2c_row = jax.lax.dot_general(
        ones, sq_c, (((1,), (1,)), ((), ())),
        preferred_element_type=jnp.float32,
        precision=jax.lax.Precision.HIGHEST)
    x2c = x2c_row[0:1, :]

    mm = jax.lax.dot_general(
        xr, xc, (((1,), (1,)), ((), ())),
        preferred_element_type=jnp.float32,
        precision=jax.lax.Precision.DEFAULT)

    dist = (x2r + x2c) - 2.0 * mm
    iota = jax.lax.broadcasted_iota(jnp.int32, (ROWS, NPTS), 1)
    offset = s * NPTS
    for k in range(KNN):
        m = jnp.min(dist, axis=1, keepdims=True)
        idx = jnp.min(jnp.where(dist == m, iota, NPTS), axis=1)
        dst_ref[0, k, :] = idx + offset
        dist = jnp.where(iota == idx[:, None], jnp.float32(jnp.inf), dist)

    row_ids = jax.lax.broadcasted_iota(jnp.int32, (KNN, ROWS), 1)
    src_ref[0] = row_ids + (r * ROWS + offset)


def kernel(x):
    n_samples, n_points, dim = x.shape
    grid = (n_samples, n_points // ROWS)
    out_shape = jax.ShapeDtypeStruct((n_samples, KNN, n_points), jnp.int32)
    dst_t, src_t = pl.pallas_call(
        _knn_kernel,
        grid=grid,
        in_specs=[
            pl.BlockSpec((1, ROWS, dim), lambda s, r: (s, r, 0)),
            pl.BlockSpec((1, n_points, dim), lambda s, r: (s, 0, 0)),
        ],
        out_specs=[
            pl.BlockSpec((1, KNN, ROWS), lambda s, r: (s, 0, r)),
            pl.BlockSpec((1, KNN, ROWS), lambda s, r: (s, 0, r)),
        ],
        out_shape=[out_shape, out_shape],
    )(x, x)
    dst = dst_t.transpose(0, 2, 1).reshape(-1)
    src = src_t.transpose(0, 2, 1).reshape(-1)
    return src, dst


# speedup vs baseline: 1.0559x; 1.0559x over previous
import jax
import jax.numpy as jnp
from jax.experimental import pallas as pl

KNN = 16
NPTS = 4096
ROWS = 128
DIM = 16


def _knn_kernel(xr_ref, xc_ref, dst_ref, src_ref):
    s = pl.program_id(0)
    r = pl.program_id(1)
    xr = xr_ref[0]
    xc = xc_ref[0]

    x2r = jnp.sum(xr * xr, axis=1, keepdims=True)
    sq_c = xc * xc
    ones = jnp.ones((8, DIM), jnp.float32)
    x2c_row = jax.lax.dot_general(
        ones, sq_c, (((1,), (1,)), ((), ())),
        preferred_element_type=jnp.float32,
        precision=jax.lax.Precision.HIGHEST)
    x2c = x2c_row[0:1, :]

    mm = jax.lax.dot_general(
        xr, xc, (((1,), (1,)), ((), ())),
        preferred_element_type=jnp.float32,
        precision=jax.lax.Precision.DEFAULT)

    dist = (x2r + x2c) - 2.0 * mm
    iota = jax.lax.broadcasted_iota(jnp.int32, (ROWS, NPTS), 1)
    offset = s * NPTS
    for k in range(KNN):
        m = jnp.min(dist, axis=1, keepdims=True)
        idx = jnp.min(jnp.where(dist == m, iota, NPTS), axis=1)
        dst_ref[0, k, :] = idx + offset
        dist = jnp.where(iota == idx[:, None], jnp.float32(jnp.inf), dist)

    row_ids = jax.lax.broadcasted_iota(jnp.int32, (KNN, ROWS), 1)
    src_ref[0] = row_ids + (r * ROWS + offset)


def kernel(x):
    n_samples, n_points, dim = x.shape
    grid = (n_samples, n_points // ROWS)
    out_shape = jax.ShapeDtypeStruct((n_samples, KNN, n_points), jnp.int32)
    dst_t, src_t = pl.pallas_call(
        _knn_kernel,
        grid=grid,
        in_specs=[
            pl.BlockSpec((1, ROWS, dim), lambda s, r: (s, r, 0)),
            pl.BlockSpec((1, n_points, dim), lambda s, r: (s, 0, 0)),
        ],
        out_specs=[
            pl.BlockSpec((1, KNN, ROWS), lambda s, r: (s, 0, r)),
            pl.BlockSpec((1, KNN, ROWS), lambda s, r: (s, 0, r)),
        ],
        out_shape=[out_shape, out_shape],
    )(x, x)
    dst = dst_t.transpose(0, 2, 1).reshape(-1)
    src = src_t.transpose(0, 2, 1).reshape(-1)
    return src, dst


# speedup vs baseline: 1.5766x; 1.4931x over previous
import jax
import jax.numpy as jnp
from jax.experimental import pallas as pl

KNN = 16
NPTS = 4096
ROWS = 256
DIM = 16
NLANE = 128
NCHUNK = NPTS // NLANE
DEPTH = 5
BIG = 2**30
INF = float('inf')


def _knn_kernel(xr_ref, xc_ref, dst_ref, src_ref):
    s = pl.program_id(0)
    r = pl.program_id(1)
    xr = xr_ref[0]
    xc = xc_ref[0]

    x2r = jnp.sum(xr * xr, axis=1, keepdims=True)
    sq_c = xc * xc
    ones = jnp.ones((8, DIM), jnp.float32)
    x2c_row = jax.lax.dot_general(
        ones, sq_c, (((1,), (1,)), ((), ())),
        preferred_element_type=jnp.float32,
        precision=jax.lax.Precision.HIGHEST)
    x2c = x2c_row[0:1, :]

    mm = jax.lax.dot_general(
        xr, xc, (((1,), (1,)), ((), ())),
        preferred_element_type=jnp.float32,
        precision=jax.lax.Precision.DEFAULT)

    dist = (x2r + x2c) - 2.0 * mm
    offset = s * NPTS

    V = [jnp.full((ROWS, NLANE), INF, jnp.float32) for _ in range(DEPTH)]
    C = [jnp.zeros((ROWS, NLANE), jnp.int32) for _ in range(DEPTH)]
    for c in range(NCHUNK):
        v = dist[:, c * NLANE:(c + 1) * NLANE]
        ci = jnp.int32(c)
        b = [v < V[k] for k in range(DEPTH)]
        newV = [jnp.where(b[0], v, V[0])]
        newC = [jnp.where(b[0], ci, C[0])]
        for k in range(1, DEPTH):
            newV.append(jnp.where(b[k - 1], V[k - 1],
                                  jnp.where(b[k], v, V[k])))
            newC.append(jnp.where(b[k - 1], C[k - 1],
                                  jnp.where(b[k], ci, C[k])))
        V, C = newV, newC

    lane_iota = jax.lax.broadcasted_iota(jnp.int32, (ROWS, NLANE), 1)
    exhausted = jnp.zeros((ROWS, NLANE), jnp.bool_)
    for k in range(KNN):
        m = jnp.min(V[0], axis=1, keepdims=True)
        jc = jnp.where(V[0] == m, C[0] * NLANE + lane_iota, BIG)
        j = jnp.min(jc, axis=1)
        dst_ref[0, k, :] = j + offset
        if k < KNN - 1:
            cond = lane_iota == (j & (NLANE - 1))[:, None]
            for q in range(DEPTH - 1):
                V[q] = jnp.where(cond, V[q + 1], V[q])
                C[q] = jnp.where(cond, C[q + 1], C[q])
            V[DEPTH - 1] = jnp.where(cond, INF, V[DEPTH - 1])
            exhausted = exhausted | (V[0] == INF)

    @pl.when(jnp.any(exhausted))
    def _fallback():
        iota = jax.lax.broadcasted_iota(jnp.int32, (ROWS, NPTS), 1)
        d = dist
        for k in range(KNN):
            m = jnp.min(d, axis=1, keepdims=True)
            idx = jnp.min(jnp.where(d == m, iota, NPTS), axis=1)
            dst_ref[0, k, :] = idx + offset
            d = jnp.where(iota == idx[:, None], INF, d)

    row_ids = jax.lax.broadcasted_iota(jnp.int32, (KNN, ROWS), 1)
    src_ref[0] = row_ids + (r * ROWS + offset)


def kernel(x):
    n_samples, n_points, dim = x.shape
    grid = (n_samples, n_points // ROWS)
    out_shape = jax.ShapeDtypeStruct((n_samples, KNN, n_points), jnp.int32)
    dst_t, src_t = pl.pallas_call(
        _knn_kernel,
        grid=grid,
        in_specs=[
            pl.BlockSpec((1, ROWS, dim), lambda s, r: (s, r, 0)),
            pl.BlockSpec((1, n_points, dim), lambda s, r: (s, 0, 0)),
        ],
        out_specs=[
            pl.BlockSpec((1, KNN, ROWS), lambda s, r: (s, 0, r)),
            pl.BlockSpec((1, KNN, ROWS), lambda s, r: (s, 0, r)),
        ],
        out_shape=[out_shape, out_shape],
    )(x, x)
    dst = dst_t.transpose(0, 2, 1).reshape(-1)
    src = src_t.transpose(0, 2, 1).reshape(-1)
    return src, dst


# speedup vs baseline: 2.3608x; 1.4974x over previous
import jax
import jax.numpy as jnp
from jax.experimental import pallas as pl

KNN = 16
NPTS = 4096
ROWS = 256
DIM = 16
NLANE = 128
NCHUNK = NPTS // NLANE
DEPTH = 5
BIG = 2**30
INF = float('inf')


def _knn_kernel(xr_ref, xc_ref, dst_ref, src_ref):
    s = pl.program_id(0)
    r = pl.program_id(1)
    xr = xr_ref[0]
    xc = xc_ref[0]

    x2r = jnp.sum(xr * xr, axis=1, keepdims=True)
    sq_c = xc * xc
    ones = jnp.ones((8, DIM), jnp.float32)
    x2c_row = jax.lax.dot_general(
        ones, sq_c, (((1,), (1,)), ((), ())),
        preferred_element_type=jnp.float32,
        precision=jax.lax.Precision.HIGHEST)
    x2c = x2c_row[0:1, :]

    mm = jax.lax.dot_general(
        xr, xc, (((1,), (1,)), ((), ())),
        preferred_element_type=jnp.float32,
        precision=jax.lax.Precision.DEFAULT)

    dist = (x2r + x2c) - 2.0 * mm
    offset = s * NPTS

    V = [jnp.full((ROWS, NLANE), INF, jnp.float32) for _ in range(DEPTH)]
    C = [jnp.zeros((ROWS, NLANE), jnp.int32) for _ in range(DEPTH)]
    for c in range(NCHUNK):
        v = dist[:, c * NLANE:(c + 1) * NLANE]
        ci = jnp.int32(c)
        b = [v < V[k] for k in range(DEPTH)]
        newV = [jnp.where(b[0], v, V[0])]
        newC = [jnp.where(b[0], ci, C[0])]
        for k in range(1, DEPTH):
            newV.append(jnp.where(b[k - 1], V[k - 1],
                                  jnp.where(b[k], v, V[k])))
            newC.append(jnp.where(b[k - 1], C[k - 1],
                                  jnp.where(b[k], ci, C[k])))
        V, C = newV, newC

    lane_iota = jax.lax.broadcasted_iota(jnp.int32, (ROWS, NLANE), 1)
    exhausted = jnp.zeros((ROWS, NLANE), jnp.bool_)
    for k in range(KNN):
        m = jnp.min(V[0], axis=1, keepdims=True)
        jc = jnp.where(V[0] == m, C[0] * NLANE + lane_iota, BIG)
        j = jnp.min(jc, axis=1)
        dst_ref[0, :, k] = j + offset
        if k < KNN - 1:
            cond = lane_iota == (j & (NLANE - 1))[:, None]
            for q in range(DEPTH - 1):
                V[q] = jnp.where(cond, V[q + 1], V[q])
                C[q] = jnp.where(cond, C[q + 1], C[q])
            V[DEPTH - 1] = jnp.where(cond, INF, V[DEPTH - 1])
            exhausted = exhausted | (V[0] == INF)

    @pl.when(jnp.any(exhausted))
    def _fallback():
        iota = jax.lax.broadcasted_iota(jnp.int32, (ROWS, NPTS), 1)
        d = dist
        for k in range(KNN):
            m = jnp.min(d, axis=1, keepdims=True)
            idx = jnp.min(jnp.where(d == m, iota, NPTS), axis=1)
            dst_ref[0, :, k] = idx + offset
            d = jnp.where(iota == idx[:, None], INF, d)

    row_ids = jax.lax.broadcasted_iota(jnp.int32, (ROWS, KNN), 0)
    src_ref[0] = row_ids + (r * ROWS + offset)


def kernel(x):
    n_samples, n_points, dim = x.shape
    grid = (n_samples, n_points // ROWS)
    out_shape = jax.ShapeDtypeStruct((n_samples, n_points, KNN), jnp.int32)
    dst_t, src_t = pl.pallas_call(
        _knn_kernel,
        grid=grid,
        in_specs=[
            pl.BlockSpec((1, ROWS, dim), lambda s, r: (s, r, 0)),
            pl.BlockSpec((1, n_points, dim), lambda s, r: (s, 0, 0)),
        ],
        out_specs=[
            pl.BlockSpec((1, ROWS, KNN), lambda s, r: (s, r, 0)),
            pl.BlockSpec((1, ROWS, KNN), lambda s, r: (s, r, 0)),
        ],
        out_shape=[out_shape, out_shape],
    )(x, x)
    dst = dst_t.reshape(-1)
    src = src_t.reshape(-1)
    return src, dst


# speedup vs baseline: 2.7583x; 1.1684x over previous
import jax
import jax.numpy as jnp
from jax.experimental import pallas as pl

KNN = 16
NPTS = 4096
ROWS = 256
DIM = 16
NLANE = 128
NCHUNK = NPTS // NLANE
DEPTH = 5
BIG = 2**30
INF = float('inf')


def _knn_kernel(xr_ref, xc_ref, dst_ref, src_ref):
    s = pl.program_id(0)
    r = pl.program_id(1)
    xr = xr_ref[0]
    xc = xc_ref[0]

    x2r = jnp.sum(xr * xr, axis=1, keepdims=True)
    sq_c = xc * xc
    ones = jnp.ones((8, DIM), jnp.float32)
    x2c_row = jax.lax.dot_general(
        ones, sq_c, (((1,), (1,)), ((), ())),
        preferred_element_type=jnp.float32,
        precision=jax.lax.Precision.HIGHEST)
    x2c = x2c_row[0:1, :]

    mm = jax.lax.dot_general(
        xr, xc, (((1,), (1,)), ((), ())),
        preferred_element_type=jnp.float32,
        precision=jax.lax.Precision.DEFAULT)

    dist = (x2r + x2c) - 2.0 * mm
    offset = s * NPTS

    V = [jnp.full((ROWS, NLANE), INF, jnp.float32) for _ in range(DEPTH)]
    C = [jnp.zeros((ROWS, NLANE), jnp.int32) for _ in range(DEPTH)]
    lane_iota = jax.lax.broadcasted_iota(jnp.int32, (ROWS, NLANE), 1)
    for c in range(NCHUNK):
        v = dist[:, c * NLANE:(c + 1) * NLANE]
        ci = lane_iota + (c * NLANE)
        b = [v < V[k] for k in range(DEPTH)]
        newV = [jnp.where(b[0], v, V[0])]
        newC = [jnp.where(b[0], ci, C[0])]
        for k in range(1, DEPTH):
            newV.append(jnp.where(b[k - 1], V[k - 1],
                                  jnp.where(b[k], v, V[k])))
            newC.append(jnp.where(b[k - 1], C[k - 1],
                                  jnp.where(b[k], ci, C[k])))
        V, C = newV, newC

    BIGF = jnp.float32(2.0**30)
    for k in range(KNN):
        m = jnp.min(V[0], axis=1, keepdims=True)
        jc = jnp.where(V[0] == m, C[0].astype(jnp.float32), BIGF)
        j = jnp.min(jc, axis=1).astype(jnp.int32)
        dst_ref[0, :, k] = j + offset
        if k < KNN - 1:
            cond = lane_iota == (j & (NLANE - 1))[:, None]
            for q in range(DEPTH - 1):
                V[q] = jnp.where(cond, V[q + 1], V[q])
                C[q] = jnp.where(cond, C[q + 1], C[q])
            V[DEPTH - 1] = jnp.where(cond, INF, V[DEPTH - 1])

    @pl.when(jnp.any(V[0] == INF))
    def _fallback():
        iota = jax.lax.broadcasted_iota(jnp.int32, (ROWS, NPTS), 1)
        d = dist
        for k in range(KNN):
            m = jnp.min(d, axis=1, keepdims=True)
            idx = jnp.min(jnp.where(d == m, iota, NPTS), axis=1)
            dst_ref[0, :, k] = idx + offset
            d = jnp.where(iota == idx[:, None], INF, d)

    row_ids = jax.lax.broadcasted_iota(jnp.int32, (ROWS, KNN), 0)
    src_ref[0] = row_ids + (r * ROWS + offset)


def kernel(x):
    n_samples, n_points, dim = x.shape
    grid = (n_samples, n_points // ROWS)
    out_shape = jax.ShapeDtypeStruct((n_samples, n_points, KNN), jnp.int32)
    dst_t, src_t = pl.pallas_call(
        _knn_kernel,
        grid=grid,
        in_specs=[
            pl.BlockSpec((1, ROWS, dim), lambda s, r: (s, r, 0)),
            pl.BlockSpec((1, n_points, dim), lambda s, r: (s, 0, 0)),
        ],
        out_specs=[
            pl.BlockSpec((1, ROWS, KNN), lambda s, r: (s, r, 0)),
            pl.BlockSpec((1, ROWS, KNN), lambda s, r: (s, r, 0)),
        ],
        out_shape=[out_shape, out_shape],
    )(x, x)
    dst = dst_t.reshape(-1)
    src = src_t.reshape(-1)
    return src, dst


# speedup vs baseline: 3.3179x; 1.2029x over previous
import jax
import jax.numpy as jnp
from jax.experimental import pallas as pl
from jax.experimental.pallas import tpu as pltpu

KNN = 16
NPTS = 4096
ROWS = 256
DIM = 16
NLANE = 128
NCHUNK = NPTS // NLANE
DEPTH = 5
BIG = 2**30
INF = float('inf')


def _knn_kernel(xr_ref, xc_ref, dst_ref, src_ref, x2c_buf):
    s = pl.program_id(0)
    r = pl.program_id(1)
    xr = xr_ref[0]
    xc = xc_ref[0]

    x2r = jnp.sum(xr * xr, axis=1, keepdims=True)

    @pl.when(r == 0)
    def _compute_x2c():
        sq_c = xc * xc
        ones = jnp.ones((8, DIM), jnp.float32)
        x2c_buf[...] = jax.lax.dot_general(
            ones, sq_c, (((1,), (1,)), ((), ())),
            preferred_element_type=jnp.float32,
            precision=jax.lax.Precision.HIGHEST)

    x2c = x2c_buf[0:1, :]

    mm = jax.lax.dot_general(
        xr, xc, (((1,), (1,)), ((), ())),
        preferred_element_type=jnp.float32,
        precision=jax.lax.Precision.DEFAULT)

    dist = (x2r + x2c) - 2.0 * mm
    offset = s * NPTS

    V = [jnp.full((ROWS, NLANE), INF, jnp.float32) for _ in range(DEPTH)]
    C = [jnp.zeros((ROWS, NLANE), jnp.float32) for _ in range(DEPTH)]
    lane_iota = jax.lax.broadcasted_iota(jnp.int32, (ROWS, NLANE), 1)
    lane_iota_f = lane_iota.astype(jnp.float32)
    for c in range(NCHUNK):
        v = dist[:, c * NLANE:(c + 1) * NLANE]
        ci = lane_iota_f + float(c * NLANE)
        b = [v < V[k] for k in range(DEPTH)]
        newV = [jnp.where(b[0], v, V[0])]
        newC = [jnp.where(b[0], ci, C[0])]
        for k in range(1, DEPTH):
            newV.append(jnp.where(b[k - 1], V[k - 1],
                                  jnp.where(b[k], v, V[k])))
            newC.append(jnp.where(b[k - 1], C[k - 1],
                                  jnp.where(b[k], ci, C[k])))
        V, C = newV, newC

    BIGF = jnp.float32(2.0**30)
    for k in range(KNN):
        m = jnp.min(V[0], axis=1, keepdims=True)
        jc = jnp.where(V[0] == m, C[0], BIGF)
        j = jnp.min(jc, axis=1).astype(jnp.int32)
        dst_ref[0, :, k] = j + offset
        if k < KNN - 1:
            cond = lane_iota == (j & (NLANE - 1))[:, None]
            for q in range(DEPTH - 1):
                V[q] = jnp.where(cond, V[q + 1], V[q])
                C[q] = jnp.where(cond, C[q + 1], C[q])
            V[DEPTH - 1] = jnp.where(cond, INF, V[DEPTH - 1])

    @pl.when(jnp.any(V[0] == INF))
    def _fallback():
        iota = jax.lax.broadcasted_iota(jnp.int32, (ROWS, NPTS), 1)
        d = dist
        for k in range(KNN):
            m = jnp.min(d, axis=1, keepdims=True)
            idx = jnp.min(jnp.where(d == m, iota, NPTS), axis=1)
            dst_ref[0, :, k] = idx + offset
            d = jnp.where(iota == idx[:, None], INF, d)

    row_ids = jax.lax.broadcasted_iota(jnp.int32, (ROWS, KNN), 0)
    src_ref[0] = row_ids + (r * ROWS + offset)


def kernel(x):
    n_samples, n_points, dim = x.shape
    grid = (n_samples, n_points // ROWS)
    out_shape = jax.ShapeDtypeStruct((n_samples, n_points, KNN), jnp.int32)
    dst_t, src_t = pl.pallas_call(
        _knn_kernel,
        grid=grid,
        in_specs=[
            pl.BlockSpec((1, ROWS, dim), lambda s, r: (s, r, 0)),
            pl.BlockSpec((1, n_points, dim), lambda s, r: (s, 0, 0)),
        ],
        out_specs=[
            pl.BlockSpec((1, ROWS, KNN), lambda s, r: (s, r, 0)),
            pl.BlockSpec((1, ROWS, KNN), lambda s, r: (s, r, 0)),
        ],
        out_shape=[out_shape, out_shape],
        scratch_shapes=[pltpu.VMEM((8, n_points), jnp.float32)],
    )(x, x)
    dst = dst_t.reshape(-1)
    src = src_t.reshape(-1)
    return src, dst
